# Initial kernel scaffold; baseline (speedup 1.0000x reference)
import jax
import jax.numpy as jnp
from jax.experimental import pallas as pl


def kernel(target, contexts, negatives, W, Cemb):
    raise NotImplementedError("write your pallas kernel here")


# speedup vs baseline: 60.6322x; 60.6322x over previous
import dataclasses

import jax
import jax.numpy as jnp
from jax import lax
from jax.experimental import pallas as pl
from jax.experimental.pallas import tpu as pltpu
from jax.experimental.pallas import tpu_sc as plsc

LANES = 16
NW = 32
VPAD = 1024


def _matmul_body(w_ref, c_ref, g_ref):
    g_ref[...] = lax.dot_general(
        w_ref[...], c_ref[...],
        dimension_numbers=(((1,), (1,)), ((), ())),
        preferred_element_type=jnp.float32)


def _make_sc_body(n_ctx, n_neg, n_chunks):
    def _sc_body(g_hbm, tgt_hbm, ctx_hbm, neg_hbm, pos_hbm, negs_hbm,
                 tgt_v, ctx_v, negi_v, rows_a, rows_b, pos_v, nego_v,
                 sem_a, sem_b):
        wid = lax.axis_index("s") * 2 + lax.axis_index("c")
        pltpu.sync_copy(tgt_hbm.at[wid], tgt_v)
        pltpu.sync_copy(ctx_hbm.at[wid], ctx_v)
        pltpu.sync_copy(neg_hbm.at[wid], negi_v)
        pltpu.async_copy(g_hbm.at[tgt_v.at[0]], rows_a, sem_a)
        lane = lax.iota(jnp.int32, LANES)

        def accum(rows, chunk):
            base = chunk * LANES
            pos_acc = jnp.zeros((LANES,), jnp.float32)
            for c in range(n_ctx):
                col = ctx_v[c, pl.ds(base, LANES)]
                pos_acc = pos_acc + plsc.load_gather(rows, [lane, col])
            neg_acc = jnp.zeros((LANES,), jnp.float32)
            for n in range(n_neg):
                col = negi_v[n, pl.ds(base, LANES)]
                neg_acc = neg_acc + plsc.load_gather(rows, [lane, col])
            pos_v[chunk, :] = pos_acc
            nego_v[chunk, :] = neg_acc

        @pl.loop(0, n_chunks // 2)
        def _(i):
            c0 = 2 * i
            pltpu.async_copy(g_hbm.at[tgt_v.at[c0 + 1]], rows_b, sem_b)
            pltpu.make_async_copy(g_hbm.at[tgt_v.at[c0]], rows_a, sem_a).wait()
            accum(rows_a, c0)

            @pl.when(i < n_chunks // 2 - 1)
            def _():
                pltpu.async_copy(g_hbm.at[tgt_v.at[c0 + 2]], rows_a, sem_a)

            pltpu.make_async_copy(g_hbm.at[tgt_v.at[c0 + 1]], rows_b,
                                  sem_b).wait()
            accum(rows_b, c0 + 1)

        pltpu.sync_copy(pos_v, pos_hbm.at[wid])
        pltpu.sync_copy(nego_v, negs_hbm.at[wid])

    return _sc_body


def _loss_body(p_ref, n_ref, o_ref):
    p = p_ref[...]
    n = n_ref[...]
    lp = jnp.maximum(-p, 0.0) + jnp.log1p(jnp.exp(-jnp.abs(p)))
    ln = jnp.maximum(n, 0.0) + jnp.log1p(jnp.exp(-jnp.abs(n)))
    o_ref[0, 0] = (jnp.sum(lp) + jnp.sum(ln)) * (1.0 / p.size)


def kernel(target, contexts, negatives, W, Cemb):
    batch, n_ctx = contexts.shape
    _, n_neg = negatives.shape
    vocab, emb = W.shape
    b_per_w = batch // NW
    n_chunks = b_per_w // LANES

    cemb_p = jnp.zeros((VPAD, emb), jnp.float32).at[:vocab].set(Cemb)
    g = pl.pallas_call(
        _matmul_body,
        out_shape=jax.ShapeDtypeStruct((vocab, VPAD), jnp.float32),
    )(W, cemb_p)

    tgt_b = target.astype(jnp.int32).reshape(NW, n_chunks, LANES)
    ctx_b = contexts.astype(jnp.int32).reshape(NW, b_per_w, n_ctx)
    ctx_b = ctx_b.transpose(0, 2, 1)
    neg_b = negatives.astype(jnp.int32).reshape(NW, b_per_w, n_neg)
    neg_b = neg_b.transpose(0, 2, 1)

    mesh = plsc.VectorSubcoreMesh(core_axis_name="c", subcore_axis_name="s",
                                  num_cores=2, num_subcores=16)
    cp = pltpu.CompilerParams()
    if "needs_layout_passes" in pltpu.CompilerParams.__dataclass_fields__:
        cp = dataclasses.replace(cp, needs_layout_passes=False)
    sc_fn = pl.kernel(
        _make_sc_body(n_ctx, n_neg, n_chunks),
        out_type=(jax.ShapeDtypeStruct((NW, n_chunks, LANES), jnp.float32),
                  jax.ShapeDtypeStruct((NW, n_chunks, LANES), jnp.float32)),
        mesh=mesh,
        scratch_types=[
            pltpu.VMEM((n_chunks, LANES), jnp.int32),
            pltpu.VMEM((n_ctx, b_per_w), jnp.int32),
            pltpu.VMEM((n_neg, b_per_w), jnp.int32),
            pltpu.VMEM((LANES, VPAD), jnp.float32),
            pltpu.VMEM((LANES, VPAD), jnp.float32),
            pltpu.VMEM((n_chunks, LANES), jnp.float32),
            pltpu.VMEM((n_chunks, LANES), jnp.float32),
            pltpu.SemaphoreType.DMA,
            pltpu.SemaphoreType.DMA,
        ],
        compiler_params=cp,
    )
    pos_b, neg_sums_b = sc_fn(g, tgt_b, ctx_b, neg_b)

    side = 128
    loss = pl.pallas_call(
        _loss_body,
        out_shape=jax.ShapeDtypeStruct((1, 1), jnp.float32),
        out_specs=pl.BlockSpec(memory_space=pltpu.SMEM),
    )(pos_b.reshape(side, side), neg_sums_b.reshape(side, side))
    return loss[0, 0]
